# baseline (speedup 1.0000x reference)
import functools

import jax
import jax.numpy as jnp
from jax import lax
from jax.experimental import pallas as pl
from jax.experimental.pallas import tpu as pltpu
from jax.experimental.pallas import tpu_sc as plsc

_NC = 2
_NS = 16
_NW = _NC * _NS
_NSLOTS = 2


def _gather_body(idx_hbm, tab_hbm, out_hbm, *refs, rows_per_w, n):
    wid = lax.axis_index("s") * _NC + lax.axis_index("c")
    base = wid * rows_per_w
    n_outer = rows_per_w // _NSLOTS

    idx_v = refs[0:_NSLOTS]
    rows_v = refs[_NSLOTS:2 * _NSLOTS]
    isem = refs[2 * _NSLOTS:3 * _NSLOTS]
    gsem = refs[3 * _NSLOTS:4 * _NSLOTS]
    osem = refs[4 * _NSLOTS:5 * _NSLOTS]

    for s in range(_NSLOTS):
        pltpu.async_copy(idx_hbm.at[base + s], idx_v[s], isem[s])

    def body(g, carry):
        j0 = base + g * _NSLOTS
        for s in range(_NSLOTS):
            j = j0 + s
            pltpu.make_async_copy(idx_hbm.at[j], idx_v[s], isem[s]).wait()

            @pl.when(g != 0)
            def _():
                pltpu.make_async_copy(
                    rows_v[s], out_hbm.at[j - _NSLOTS], osem[s]).wait()

            pltpu.async_copy(tab_hbm.at[idx_v[s]], rows_v[s], gsem[s])
        for s in range(_NSLOTS):
            j = j0 + s
            pltpu.make_async_copy(tab_hbm.at[idx_v[s]], rows_v[s], gsem[s]).wait()
            pltpu.async_copy(rows_v[s], out_hbm.at[j], osem[s])

            @pl.when(g != n_outer - 1)
            def _():
                pltpu.async_copy(idx_hbm.at[j + _NSLOTS], idx_v[s], isem[s])

        return carry

    lax.fori_loop(0, n_outer, body, 0)

    last = base + rows_per_w - _NSLOTS
    for s in range(_NSLOTS):
        pltpu.make_async_copy(rows_v[s], out_hbm.at[last + s], osem[s]).wait()


def kernel(sparse_input, table):
    B, N = sparse_input.shape
    V, D = table.shape
    assert B % (_NW * _NSLOTS) == 0
    rows_per_w = B // _NW

    mesh = plsc.VectorSubcoreMesh(core_axis_name="c", subcore_axis_name="s")

    run = functools.partial(
        pl.kernel,
        out_type=jax.ShapeDtypeStruct((B, N, D), jnp.float32),
        mesh=mesh,
        scratch_types=(
            [pltpu.VMEM((N,), jnp.int32)] * _NSLOTS
            + [pltpu.VMEM((N, D), jnp.float32)] * _NSLOTS
            + [pltpu.SemaphoreType.DMA] * (3 * _NSLOTS)
        ),
        compiler_params=pltpu.CompilerParams(
            use_tc_tiling_on_sc=False, disable_bounds_checks=True),
    )(functools.partial(_gather_body, rows_per_w=rows_per_w, n=N))

    return run(sparse_input, table)


# speedup vs baseline: 1.1910x; 1.1910x over previous
import functools

import jax
import jax.numpy as jnp
from jax import lax
from jax.experimental import pallas as pl
from jax.experimental.pallas import tpu as pltpu
from jax.experimental.pallas import tpu_sc as plsc

_NC = 2
_NS = 16
_NW = _NC * _NS
_CHUNK = 2048
_NSLOTS = 2


def _gather_body(idx_hbm, tab_hbm, out_hbm, *refs, per_w, n_chunks):
    wid = lax.axis_index("s") * _NC + lax.axis_index("c")
    base = wid * per_w
    n_outer = n_chunks // _NSLOTS

    idx_v = refs[0:_NSLOTS]
    rows_v = refs[_NSLOTS:2 * _NSLOTS]
    isem = refs[2 * _NSLOTS:3 * _NSLOTS]
    gsem = refs[3 * _NSLOTS:4 * _NSLOTS]
    osem = refs[4 * _NSLOTS:5 * _NSLOTS]

    def idx_slice(j):
        return idx_hbm.at[pl.ds(base + j * _CHUNK, _CHUNK)]

    def out_slice(j):
        return out_hbm.at[pl.ds(base + j * _CHUNK, _CHUNK)]

    for s in range(_NSLOTS):
        pltpu.async_copy(idx_slice(s), idx_v[s], isem[s])

    def body(g, carry):
        j0 = g * _NSLOTS
        for s in range(_NSLOTS):
            j = j0 + s
            pltpu.make_async_copy(idx_slice(j), idx_v[s], isem[s]).wait()

            @pl.when(g != 0)
            def _():
                pltpu.make_async_copy(
                    rows_v[s], out_slice(j - _NSLOTS), osem[s]).wait()

            pltpu.async_copy(tab_hbm.at[idx_v[s]], rows_v[s], gsem[s])
        for s in range(_NSLOTS):
            j = j0 + s
            pltpu.make_async_copy(tab_hbm.at[idx_v[s]], rows_v[s], gsem[s]).wait()
            pltpu.async_copy(rows_v[s], out_slice(j), osem[s])

            @pl.when(g != n_outer - 1)
            def _():
                pltpu.async_copy(idx_slice(j + _NSLOTS), idx_v[s], isem[s])

        return carry

    lax.fori_loop(0, n_outer, body, 0)

    last = n_chunks - _NSLOTS
    for s in range(_NSLOTS):
        pltpu.make_async_copy(rows_v[s], out_slice(last + s), osem[s]).wait()


def _pack_body(mid_hbm, out_hbm, *refs, groups_per_w, n, d):
    wid = lax.axis_index("s") * _NC + lax.axis_index("c")
    base = wid * groups_per_w
    vpw = 128 // d

    buf_in = refs[0:2]
    buf_out = refs[2:4]
    isem = refs[4:6]
    osem = refs[6:8]

    for si in range(2):
        pltpu.async_copy(mid_hbm.at[pl.ds(base + si, 1)], buf_in[si], isem[si])

    n_outer = groups_per_w // 2

    def body(go, carry):
        for si in range(2):
            g = go * 2 + si
            gi = base + g
            pltpu.make_async_copy(
                mid_hbm.at[pl.ds(gi, 1)], buf_in[si], isem[si]).wait()

            for r8 in range(8):
                b = gi * 8 + r8
                s = r8 & 1

                if si == 1 or r8 >= 2:
                    pltpu.make_async_copy(
                        buf_out[s], out_hbm.at[pl.ds(b - 2, 1)], osem[s]).wait()
                else:
                    @pl.when(go != 0)
                    def _():
                        pltpu.make_async_copy(
                            buf_out[s], out_hbm.at[pl.ds(b - 2, 1)], osem[s]).wait()

                def srow(u0, carry2):
                    r_in = r8 * (n // vpw) + u0
                    for v in range(vpw):
                        buf_out[s][0, u0 * vpw + v, :] = \
                            buf_in[si][0, r_in, pl.ds(v * d, d)]
                    return carry2

                lax.fori_loop(0, n // vpw, srow, 0)
                pltpu.async_copy(buf_out[s], out_hbm.at[pl.ds(b, 1)], osem[s])

            @pl.when(go != n_outer - 1)
            def _():
                pltpu.async_copy(
                    mid_hbm.at[pl.ds(gi + 2, 1)], buf_in[si], isem[si])

        return carry

    lax.fori_loop(0, n_outer, body, 0)

    last = (base + groups_per_w) * 8
    for s in range(2):
        pltpu.make_async_copy(buf_out[s], out_hbm.at[pl.ds(last - 2 + s, 1)], osem[s]).wait()


def kernel(sparse_input, table):
    B, N = sparse_input.shape
    V, D = table.shape
    total = B * N
    assert total % (_NW * _CHUNK * _NSLOTS) == 0
    per_w = total // _NW
    n_chunks = per_w // _CHUNK

    flat_idx = sparse_input.reshape(total)
    mesh = plsc.VectorSubcoreMesh(core_axis_name="c", subcore_axis_name="s")

    gather_run = functools.partial(
        pl.kernel,
        out_type=jax.ShapeDtypeStruct((total, D), jnp.float32),
        mesh=mesh,
        scratch_types=(
            [pltpu.VMEM((_CHUNK,), jnp.int32)] * _NSLOTS
            + [pltpu.VMEM((_CHUNK, D), jnp.float32)] * _NSLOTS
            + [pltpu.SemaphoreType.DMA] * (3 * _NSLOTS)
        ),
        compiler_params=pltpu.CompilerParams(
            use_tc_tiling_on_sc=False, disable_bounds_checks=True),
    )(functools.partial(_gather_body, per_w=per_w, n_chunks=n_chunks))

    wpg = (8 * N * D) // 128
    mid3 = gather_run(flat_idx, table).reshape(B // 8, wpg, 128)

    groups_per_w = B // (8 * _NW)
    pack_run = functools.partial(
        pl.kernel,
        out_type=jax.ShapeDtypeStruct((B, N, D), jnp.float32),
        mesh=mesh,
        scratch_types=(
            [pltpu.VMEM((1, wpg, 128), jnp.float32)] * 2
            + [pltpu.VMEM((1, N, D), jnp.float32)] * 2
            + [pltpu.SemaphoreType.DMA] * 4
        ),
        compiler_params=pltpu.CompilerParams(
            use_tc_tiling_on_sc=True, disable_bounds_checks=True),
    )(functools.partial(_pack_body, groups_per_w=groups_per_w, n=N, d=D))

    return pack_run(mid3)
